# baseline (speedup 1.0000x reference)
import functools

import jax
import jax.numpy as jnp
from jax import lax
from jax.experimental import pallas as pl
from jax.experimental.pallas import tpu as pltpu
from jax.experimental.pallas import tpu_sc as plsc

N = 256
B = 4
C = 128
TBL = 64
NC, NS = 2, 16
NW = NC * NS
ROWS = N * N * B
RPW = ROWS // NW
CHUNK = 128
NCH = RPW // CHUNK


def _prep_kernel(i0_ref, i1_ref, i0f_ref, i1f_ref, p0_ref, p1_ref, co_ref,
                 table_ref, idx_ref, cm_ref):
    zrow = jnp.zeros((1, C), jnp.float32)
    p0p = jnp.concatenate([p0_ref[...], zrow], axis=0)
    p1p = jnp.concatenate([p1_ref[...], zrow], axis=0)
    table_ref[...] = p0p[:, None, :] + p1p[None, :, :]

    co0 = co_ref[0]
    co1 = co_ref[1]
    q = lax.broadcasted_iota(jnp.int32, (N, N * B), 1)
    bat = q & (B - 1)

    def xterm(ref):
        a = jnp.broadcast_to(ref[:, 0:1], (N, N * B))
        for k in range(1, B):
            a = jnp.where(bat == k, ref[:, k:k + 1], a)
        return a

    a0 = xterm(i0_ref)
    a1 = xterm(i1_ref)
    r0 = a0 - i0f_ref[...] + co0
    r1 = a1 - i1f_ref[...] + co1
    cm_ref[...] = jnp.where((r0 < 0) | (r1 < 0), jnp.int32(1), jnp.int32(0))
    idx_ref[...] = jnp.maximum(r0, 0) * TBL + jnp.maximum(r1, 0)


def _prep(i0, i1, i0f, i1f, p0, p1, center_offset):
    return pl.pallas_call(
        _prep_kernel,
        in_specs=[pl.BlockSpec(memory_space=pltpu.VMEM)] * 6 +
                 [pl.BlockSpec(memory_space=pltpu.SMEM)],
        out_specs=[
            pl.BlockSpec(memory_space=pltpu.VMEM),
            pl.BlockSpec(memory_space=pltpu.VMEM),
            pl.BlockSpec(memory_space=pltpu.VMEM),
        ],
        out_shape=[
            jax.ShapeDtypeStruct((TBL, TBL, C), jnp.float32),
            jax.ShapeDtypeStruct((N, N * B), jnp.int32),
            jax.ShapeDtypeStruct((N, N * B), jnp.int32),
        ],
    )(i0, i1, i0f, i1f, p0, p1, center_offset)


NBUF = 4


def _gather_body(table_hbm, idx_hbm, out_hbm, idx_v, rows_v, tbl_sh, gsem, psem):
    sid = lax.axis_index("s")
    wid = sid * NC + lax.axis_index("c")
    @pl.when(sid == 0)
    def _():
        pltpu.sync_copy(table_hbm, tbl_sh)
    pltpu.sync_copy(idx_hbm.at[wid], idx_v)
    plsc.subcore_barrier()
    base = wid * RPW

    def gather(j, buf, jmod):
        src = table_hbm if jmod == NBUF - 1 else tbl_sh
        return pltpu.make_async_copy(
            src.at[idx_v.at[j]], rows_v.at[buf], gsem.at[buf])

    def put(j, buf):
        return pltpu.make_async_copy(
            rows_v.at[buf], out_hbm.at[pl.ds(base + j * CHUNK, CHUNK)],
            psem.at[buf])

    for k in range(NBUF - 1):
        gather(k, k, k % NBUF).start()

    def body(g, _):
        for k in range(NBUF):
            j = g * NBUF + k
            gather(j, k, k).wait()
            put(j, k).start()
            nj = j + NBUF - 1
            nbuf = (k + NBUF - 1) % NBUF

            @pl.when(nj < NCH)
            def _():
                @pl.when(j >= 1)
                def _():
                    put(j - 1, nbuf).wait()

                gather(nj, nbuf, nbuf).start()

        return 0

    lax.fori_loop(0, NCH // NBUF, body, 0)
    put(NCH - 1, (NCH - 1) % NBUF).wait()


@functools.partial(
    pl.kernel,
    mesh=plsc.VectorSubcoreMesh(core_axis_name="c", subcore_axis_name="s"),
    out_type=jax.ShapeDtypeStruct((ROWS, C), jnp.float32),
    scratch_types=[
        pltpu.VMEM((NCH, CHUNK), jnp.int32),
        pltpu.VMEM((NBUF, CHUNK, C), jnp.float32),
        pltpu.VMEM_SHARED((TBL * TBL, C), jnp.float32),
        pltpu.SemaphoreType.DMA((NBUF,)),
        pltpu.SemaphoreType.DMA((NBUF,)),
    ],
)
def _sc_gather(table_hbm, idx_hbm, out_hbm, idx_v, rows_v, tbl_sh, gsem, psem):
    _gather_body(table_hbm, idx_hbm, out_hbm, idx_v, rows_v, tbl_sh, gsem, psem)


def kernel(i, p0, p1, center_offset):
    i = i.astype(jnp.int32)
    center_offset = center_offset.astype(jnp.int32)
    i0 = i[:, :, 0]
    i1 = i[:, :, 1]
    i0f = i0.reshape(1, N * B)
    i1f = i1.reshape(1, N * B)
    table, idx, cm = _prep(i0, i1, i0f, i1f, p0, p1, center_offset)
    table = table.reshape(TBL * TBL, C)
    idx = idx.reshape(NW, NCH, CHUNK)
    pe = _sc_gather(table, idx)
    pe = pe.reshape(N, N, B, C)
    cm = cm.reshape(N, N, B).astype(jnp.bool_)
    return pe, cm


# speedup vs baseline: 1.2885x; 1.2885x over previous
import functools

import jax
import jax.numpy as jnp
from jax import lax
from jax.experimental import pallas as pl
from jax.experimental.pallas import tpu as pltpu
from jax.experimental.pallas import tpu_sc as plsc

N = 256
B = 4
C = 128
TBL = 64
NC, NS = 2, 16
NW = NC * NS
ROWS = N * N * B
RPW = ROWS // NW
CHUNK = 128
NCH = RPW // CHUNK
XPW = N // NW


def _prep_kernel(i0_ref, i1_ref, i0f_ref, i1f_ref, p0_ref, p1_ref, co_ref,
                 table_ref, idx_ref, cm_ref):
    zrow = jnp.zeros((1, C), jnp.float32)
    p0p = jnp.concatenate([p0_ref[...], zrow], axis=0)
    p1p = jnp.concatenate([p1_ref[...], zrow], axis=0)
    table_ref[...] = (p0p[:, None, :] + p1p[None, :, :]).reshape(TBL * TBL, C)

    co0 = co_ref[0]
    co1 = co_ref[1]
    q = lax.broadcasted_iota(jnp.int32, (N, N * B), 1)
    bat = q & (B - 1)

    def xterm(ref):
        a = jnp.broadcast_to(ref[:, 0:1], (N, N * B))
        for k in range(1, B):
            a = jnp.where(bat == k, ref[:, k:k + 1], a)
        return a

    a0 = xterm(i0_ref)
    a1 = xterm(i1_ref)
    r0 = a0 - i0f_ref[...] + co0
    r1 = a1 - i1f_ref[...] + co1
    cm_ref[...] = (r0 < 0) | (r1 < 0)
    idx_ref[...] = jnp.maximum(r0, 0) * TBL + jnp.maximum(r1, 0)


def _prep(i0, i1, i0f, i1f, p0, p1, center_offset):
    return pl.pallas_call(
        _prep_kernel,
        in_specs=[pl.BlockSpec(memory_space=pltpu.VMEM)] * 6 +
                 [pl.BlockSpec(memory_space=pltpu.SMEM)],
        out_specs=[
            pl.BlockSpec(memory_space=pltpu.VMEM),
            pl.BlockSpec(memory_space=pltpu.VMEM),
            pl.BlockSpec(memory_space=pltpu.VMEM),
        ],
        out_shape=[
            jax.ShapeDtypeStruct((TBL * TBL, C), jnp.float32),
            jax.ShapeDtypeStruct((N, N * B), jnp.int32),
            jax.ShapeDtypeStruct((N, N * B), jnp.bool_),
        ],
    )(i0, i1, i0f, i1f, p0, p1, center_offset)


NBUF = 4


def _gather_body(table_hbm, idx_hbm, out_hbm, idx_v, rows_v, tbl_sh, gsem, psem):
    sid = lax.axis_index("s")
    wid = sid * NC + lax.axis_index("c")
    @pl.when(sid == 0)
    def _():
        pltpu.sync_copy(table_hbm, tbl_sh)
    pltpu.sync_copy(idx_hbm.at[pl.ds(wid * XPW, XPW)], idx_v)
    plsc.subcore_barrier()
    base = wid * RPW
    CPX = N * B // CHUNK

    def gather(j, buf, jmod):
        return pltpu.make_async_copy(
            tbl_sh.at[idx_v.at[j // CPX, j % CPX]], rows_v.at[buf],
            gsem.at[buf])

    def put(j, buf):
        return pltpu.make_async_copy(
            rows_v.at[buf], out_hbm.at[pl.ds(base + j * CHUNK, CHUNK)],
            psem.at[buf])

    for k in range(NBUF - 1):
        gather(k, k, k % NBUF).start()

    def body(g, _):
        for k in range(NBUF):
            j = g * NBUF + k
            gather(j, k, k).wait()
            put(j, k).start()
            nj = j + NBUF - 1
            nbuf = (k + NBUF - 1) % NBUF

            @pl.when(nj < NCH)
            def _():
                @pl.when(j >= 1)
                def _():
                    put(j - 1, nbuf).wait()

                gather(nj, nbuf, nbuf).start()

        return 0

    lax.fori_loop(0, NCH // NBUF, body, 0)
    put(NCH - 1, (NCH - 1) % NBUF).wait()


@functools.partial(
    pl.kernel,
    mesh=plsc.VectorSubcoreMesh(core_axis_name="c", subcore_axis_name="s"),
    out_type=jax.ShapeDtypeStruct((ROWS, C), jnp.float32),
    scratch_types=[
        pltpu.VMEM((XPW, N * B // CHUNK, CHUNK), jnp.int32),
        pltpu.VMEM((NBUF, CHUNK, C), jnp.float32),
        pltpu.VMEM_SHARED((TBL * TBL, C), jnp.float32),
        pltpu.SemaphoreType.DMA((NBUF,)),
        pltpu.SemaphoreType.DMA((NBUF,)),
    ],
)
def _sc_gather(table_hbm, idx_hbm, out_hbm, idx_v, rows_v, tbl_sh, gsem, psem):
    _gather_body(table_hbm, idx_hbm, out_hbm, idx_v, rows_v, tbl_sh, gsem, psem)


def kernel(i, p0, p1, center_offset):
    i = i.astype(jnp.int32)
    center_offset = center_offset.astype(jnp.int32)
    i0 = i[:, :, 0]
    i1 = i[:, :, 1]
    i0f = i0.reshape(1, N * B)
    i1f = i1.reshape(1, N * B)
    table, idx, cm = _prep(i0, i1, i0f, i1f, p0, p1, center_offset)
    idx = idx.reshape(N, N * B // CHUNK, CHUNK)
    pe = _sc_gather(table, idx)
    pe = pe.reshape(N, N, B, C)
    cm = cm.reshape(N, N, B)
    return pe, cm


# speedup vs baseline: 1.3241x; 1.0276x over previous
import functools

import jax
import jax.numpy as jnp
from jax import lax
from jax.experimental import pallas as pl
from jax.experimental.pallas import tpu as pltpu
from jax.experimental.pallas import tpu_sc as plsc

N = 256
B = 4
C = 128
TBL = 64
NC, NS = 2, 16
NW = NC * NS
ROWS = N * N * B
RPW = ROWS // NW
CHUNK = 128
NCH = RPW // CHUNK
XPW = N // NW


def _prep_kernel(i0_ref, i1_ref, i0f_ref, i1f_ref, p0_ref, p1_ref, co_ref,
                 table_ref, idx_ref, cm_ref):
    zrow = jnp.zeros((1, C), jnp.float32)
    p0p = jnp.concatenate([p0_ref[...], zrow], axis=0)
    p1p = jnp.concatenate([p1_ref[...], zrow], axis=0)
    table_ref[...] = (p0p[:, None, :] + p1p[None, :, :]).reshape(TBL * TBL, C)

    co0 = co_ref[0]
    co1 = co_ref[1]
    q = lax.broadcasted_iota(jnp.int32, (N, N * B), 1)
    bat = q & (B - 1)

    def xterm(ref):
        a = jnp.broadcast_to(ref[:, 0:1], (N, N * B))
        for k in range(1, B):
            a = jnp.where(bat == k, ref[:, k:k + 1], a)
        return a

    a0 = xterm(i0_ref)
    a1 = xterm(i1_ref)
    r0 = a0 - i0f_ref[...] + co0
    r1 = a1 - i1f_ref[...] + co1
    cm_ref[...] = (r0 < 0) | (r1 < 0)
    idx_ref[...] = jnp.maximum(r0, 0) * TBL + jnp.maximum(r1, 0)


def _prep(i0, i1, i0f, i1f, p0, p1, center_offset):
    return pl.pallas_call(
        _prep_kernel,
        in_specs=[pl.BlockSpec(memory_space=pltpu.VMEM)] * 6 +
                 [pl.BlockSpec(memory_space=pltpu.SMEM)],
        out_specs=[
            pl.BlockSpec(memory_space=pltpu.VMEM),
            pl.BlockSpec(memory_space=pltpu.VMEM),
            pl.BlockSpec(memory_space=pltpu.VMEM),
        ],
        out_shape=[
            jax.ShapeDtypeStruct((TBL * TBL, C), jnp.float32),
            jax.ShapeDtypeStruct((N, N * B), jnp.int32),
            jax.ShapeDtypeStruct((N, N * B), jnp.bool_),
        ],
    )(i0, i1, i0f, i1f, p0, p1, center_offset)


NBUF = 4


def _gather_body(table_hbm, idx_hbm, out_hbm, idx_v, rows_v, tbl_sh, gsem, psem):
    sid = lax.axis_index("s")
    wid = sid * NC + lax.axis_index("c")
    @pl.when(sid == 0)
    def _():
        pltpu.sync_copy(table_hbm, tbl_sh)
    pltpu.sync_copy(idx_hbm.at[pl.ds(wid * XPW, XPW)], idx_v)
    plsc.subcore_barrier()
    base = wid * RPW
    CPX = N * B // CHUNK

    def gather(j, buf, jmod):
        return pltpu.make_async_copy(
            tbl_sh.at[idx_v.at[j // CPX, pl.ds((j % CPX) * CHUNK, CHUNK)]],
            rows_v.at[buf], gsem.at[buf])

    def put(j, buf):
        return pltpu.make_async_copy(
            rows_v.at[buf], out_hbm.at[pl.ds(base + j * CHUNK, CHUNK)],
            psem.at[buf])

    for k in range(NBUF - 1):
        gather(k, k, k % NBUF).start()

    def body(g, _):
        for k in range(NBUF):
            j = g * NBUF + k
            gather(j, k, k).wait()
            put(j, k).start()
            nj = j + NBUF - 1
            nbuf = (k + NBUF - 1) % NBUF

            @pl.when(nj < NCH)
            def _():
                @pl.when(j >= 1)
                def _():
                    put(j - 1, nbuf).wait()

                gather(nj, nbuf, nbuf).start()

        return 0

    lax.fori_loop(0, NCH // NBUF, body, 0)
    put(NCH - 1, (NCH - 1) % NBUF).wait()


@functools.partial(
    pl.kernel,
    mesh=plsc.VectorSubcoreMesh(core_axis_name="c", subcore_axis_name="s"),
    out_type=jax.ShapeDtypeStruct((ROWS, C), jnp.float32),
    scratch_types=[
        pltpu.VMEM((XPW, N * B), jnp.int32),
        pltpu.VMEM((NBUF, CHUNK, C), jnp.float32),
        pltpu.VMEM_SHARED((TBL * TBL, C), jnp.float32),
        pltpu.SemaphoreType.DMA((NBUF,)),
        pltpu.SemaphoreType.DMA((NBUF,)),
    ],
)
def _sc_gather(table_hbm, idx_hbm, out_hbm, idx_v, rows_v, tbl_sh, gsem, psem):
    _gather_body(table_hbm, idx_hbm, out_hbm, idx_v, rows_v, tbl_sh, gsem, psem)


def kernel(i, p0, p1, center_offset):
    i = i.astype(jnp.int32)
    center_offset = center_offset.astype(jnp.int32)
    i0 = i[:, :, 0]
    i1 = i[:, :, 1]
    i0f = i0.reshape(1, N * B)
    i1f = i1.reshape(1, N * B)
    table, idx, cm = _prep(i0, i1, i0f, i1f, p0, p1, center_offset)
    pe = _sc_gather(table, idx)
    pe = pe.reshape(N, N, B, C)
    cm = cm.reshape(N, N, B)
    return pe, cm


# speedup vs baseline: 1.3361x; 1.0091x over previous
import functools

import jax
import jax.numpy as jnp
from jax import lax
from jax.experimental import pallas as pl
from jax.experimental.pallas import tpu as pltpu
from jax.experimental.pallas import tpu_sc as plsc

N = 256
B = 4
C = 128
TBL = 64
NC, NS = 2, 16
NW = NC * NS
ROWS = N * N * B
RPW = ROWS // NW
CHUNK = 128
NCH = RPW // CHUNK
XPW = N // NW


def _rel(i0_ref, i1_ref, i0f_ref, i1f_ref, co_ref):
    q = lax.broadcasted_iota(jnp.int32, (N, N * B), 1)
    bat = q & (B - 1)

    def xterm(ref):
        a = jnp.broadcast_to(ref[:, 0:1], (N, N * B))
        for k in range(1, B):
            a = jnp.where(bat == k, ref[:, k:k + 1], a)
        return a

    r0 = xterm(i0_ref) - i0f_ref[...] + co_ref[0]
    r1 = xterm(i1_ref) - i1f_ref[...] + co_ref[1]
    return r0, r1


def _prep_kernel(i0_ref, i1_ref, i0f_ref, i1f_ref, p0_ref, p1_ref, co_ref,
                 table_ref, idx_ref):
    zrow = jnp.zeros((1, C), jnp.float32)
    p0p = jnp.concatenate([p0_ref[...], zrow], axis=0)
    p1p = jnp.concatenate([p1_ref[...], zrow], axis=0)
    table_ref[...] = (p0p[:, None, :] + p1p[None, :, :]).reshape(TBL * TBL, C)
    r0, r1 = _rel(i0_ref, i1_ref, i0f_ref, i1f_ref, co_ref)
    idx_ref[...] = jnp.maximum(r0, 0) * TBL + jnp.maximum(r1, 0)


def _cm_kernel(i0_ref, i1_ref, i0f_ref, i1f_ref, co_ref, cm_ref):
    r0, r1 = _rel(i0_ref, i1_ref, i0f_ref, i1f_ref, co_ref)
    cm_ref[...] = (r0 < 0) | (r1 < 0)


def _prep(i0, i1, i0f, i1f, p0, p1, center_offset):
    return pl.pallas_call(
        _prep_kernel,
        in_specs=[pl.BlockSpec(memory_space=pltpu.VMEM)] * 6 +
                 [pl.BlockSpec(memory_space=pltpu.SMEM)],
        out_specs=[
            pl.BlockSpec(memory_space=pltpu.VMEM),
            pl.BlockSpec(memory_space=pltpu.VMEM),
        ],
        out_shape=[
            jax.ShapeDtypeStruct((TBL * TBL, C), jnp.float32),
            jax.ShapeDtypeStruct((N, N * B), jnp.int32),
        ],
    )(i0, i1, i0f, i1f, p0, p1, center_offset)


def _cm(i0, i1, i0f, i1f, center_offset):
    return pl.pallas_call(
        _cm_kernel,
        in_specs=[pl.BlockSpec(memory_space=pltpu.VMEM)] * 4 +
                 [pl.BlockSpec(memory_space=pltpu.SMEM)],
        out_specs=pl.BlockSpec(memory_space=pltpu.VMEM),
        out_shape=jax.ShapeDtypeStruct((N, N * B), jnp.bool_),
    )(i0, i1, i0f, i1f, center_offset)


NBUF = 4


def _gather_body(table_hbm, idx_hbm, out_hbm, idx_v, rows_v, tbl_sh, gsem, psem):
    sid = lax.axis_index("s")
    wid = sid * NC + lax.axis_index("c")
    @pl.when(sid == 0)
    def _():
        pltpu.sync_copy(table_hbm, tbl_sh)
    pltpu.sync_copy(idx_hbm.at[pl.ds(wid * XPW, XPW)], idx_v)
    plsc.subcore_barrier()
    base = wid * RPW
    CPX = N * B // CHUNK

    def gather(j, buf, jmod):
        return pltpu.make_async_copy(
            tbl_sh.at[idx_v.at[j // CPX, pl.ds((j % CPX) * CHUNK, CHUNK)]],
            rows_v.at[buf], gsem.at[buf])

    def put(j, buf):
        return pltpu.make_async_copy(
            rows_v.at[buf], out_hbm.at[pl.ds(base + j * CHUNK, CHUNK)],
            psem.at[buf])

    for k in range(NBUF - 1):
        gather(k, k, k % NBUF).start()

    def body(g, _):
        for k in range(NBUF):
            j = g * NBUF + k
            gather(j, k, k).wait()
            put(j, k).start()
            nj = j + NBUF - 1
            nbuf = (k + NBUF - 1) % NBUF

            @pl.when(nj < NCH)
            def _():
                @pl.when(j >= 1)
                def _():
                    put(j - 1, nbuf).wait()

                gather(nj, nbuf, nbuf).start()

        return 0

    lax.fori_loop(0, NCH // NBUF, body, 0)
    put(NCH - 1, (NCH - 1) % NBUF).wait()


@functools.partial(
    pl.kernel,
    mesh=plsc.VectorSubcoreMesh(core_axis_name="c", subcore_axis_name="s"),
    out_type=jax.ShapeDtypeStruct((ROWS, C), jnp.float32),
    scratch_types=[
        pltpu.VMEM((XPW, N * B), jnp.int32),
        pltpu.VMEM((NBUF, CHUNK, C), jnp.float32),
        pltpu.VMEM_SHARED((TBL * TBL, C), jnp.float32),
        pltpu.SemaphoreType.DMA((NBUF,)),
        pltpu.SemaphoreType.DMA((NBUF,)),
    ],
)
def _sc_gather(table_hbm, idx_hbm, out_hbm, idx_v, rows_v, tbl_sh, gsem, psem):
    _gather_body(table_hbm, idx_hbm, out_hbm, idx_v, rows_v, tbl_sh, gsem, psem)


def kernel(i, p0, p1, center_offset):
    i = i.astype(jnp.int32)
    center_offset = center_offset.astype(jnp.int32)
    i0 = i[:, :, 0]
    i1 = i[:, :, 1]
    i0f = i0.reshape(1, N * B)
    i1f = i1.reshape(1, N * B)
    table, idx = _prep(i0, i1, i0f, i1f, p0, p1, center_offset)
    cm = _cm(i0, i1, i0f, i1f, center_offset)
    pe = _sc_gather(table, idx)
    pe = pe.reshape(N, N, B, C)
    cm = cm.reshape(N, N, B)
    return pe, cm


# speedup vs baseline: 1.3382x; 1.0016x over previous
import functools

import jax
import jax.numpy as jnp
from jax import lax
from jax.experimental import pallas as pl
from jax.experimental.pallas import tpu as pltpu
from jax.experimental.pallas import tpu_sc as plsc

N = 256
B = 4
C = 128
TBL = 64
NC, NS = 2, 16
NW = NC * NS
ROWS = N * N * B
RPW = ROWS // NW
CHUNK = 128
NCH = RPW // CHUNK
XPW = N // NW


def _rel(i0_ref, i1_ref, i0f_ref, i1f_ref, co_ref):
    q = lax.broadcasted_iota(jnp.int32, (N, N * B), 1)
    bat = q & (B - 1)

    def xterm(ref):
        a = jnp.broadcast_to(ref[:, 0:1], (N, N * B))
        for k in range(1, B):
            a = jnp.where(bat == k, ref[:, k:k + 1], a)
        return a

    r0 = xterm(i0_ref) - i0f_ref[...] + co_ref[0]
    r1 = xterm(i1_ref) - i1f_ref[...] + co_ref[1]
    return r0, r1


def _prep_kernel(i0_ref, i1_ref, i0f_ref, i1f_ref, p0_ref, p1_ref, co_ref,
                 table_ref, idx_ref):
    zrow = jnp.zeros((1, C), jnp.float32)
    p0p = jnp.concatenate([p0_ref[...], zrow], axis=0)
    p1p = jnp.concatenate([p1_ref[...], zrow], axis=0)
    table_ref[...] = (p0p[:, None, :] + p1p[None, :, :]).reshape(TBL * TBL, C)
    r0, r1 = _rel(i0_ref, i1_ref, i0f_ref, i1f_ref, co_ref)
    idx_ref[...] = jnp.maximum(r0, 0) * TBL + jnp.maximum(r1, 0)


def _cm_kernel(i0_ref, i1_ref, i0f_ref, i1f_ref, co_ref, cm_ref):
    r0, r1 = _rel(i0_ref, i1_ref, i0f_ref, i1f_ref, co_ref)
    cm_ref[...] = (r0 < 0) | (r1 < 0)


def _prep(i0, i1, i0f, i1f, p0, p1, center_offset):
    return pl.pallas_call(
        _prep_kernel,
        in_specs=[pl.BlockSpec(memory_space=pltpu.VMEM)] * 6 +
                 [pl.BlockSpec(memory_space=pltpu.SMEM)],
        out_specs=[
            pl.BlockSpec(memory_space=pltpu.VMEM),
            pl.BlockSpec(memory_space=pltpu.VMEM),
        ],
        out_shape=[
            jax.ShapeDtypeStruct((TBL * TBL, C), jnp.float32),
            jax.ShapeDtypeStruct((N, N * B), jnp.int32),
        ],
    )(i0, i1, i0f, i1f, p0, p1, center_offset)


def _cm(i0, i1, i0f, i1f, center_offset):
    return pl.pallas_call(
        _cm_kernel,
        in_specs=[pl.BlockSpec(memory_space=pltpu.VMEM)] * 4 +
                 [pl.BlockSpec(memory_space=pltpu.SMEM)],
        out_specs=pl.BlockSpec(memory_space=pltpu.VMEM),
        out_shape=jax.ShapeDtypeStruct((N, N * B), jnp.bool_),
    )(i0, i1, i0f, i1f, center_offset)


NBUF = 4


def _gather_body(table_hbm, idx_hbm, out_hbm, idx_v, rows_v, tbl_sh, gsem, psem):
    sid = lax.axis_index("s")
    wid = sid * NC + lax.axis_index("c")
    @pl.when(sid == 0)
    def _():
        pltpu.sync_copy(table_hbm, tbl_sh)
    pltpu.sync_copy(idx_hbm.at[pl.ds(wid * XPW, XPW)], idx_v)
    plsc.subcore_barrier()
    base = wid * RPW
    CPX = N * B // CHUNK

    def gather(j, buf):
        return pltpu.make_async_copy(
            tbl_sh.at[idx_v.at[j // CPX, pl.ds((j % CPX) * CHUNK, CHUNK)]],
            rows_v.at[buf], gsem.at[buf])

    def put(j, buf):
        return pltpu.make_async_copy(
            rows_v.at[buf], out_hbm.at[pl.ds(base + j * CHUNK, CHUNK)],
            psem.at[buf])

    for k in range(NBUF - 1):
        gather(k, k).start()

    def body(g, _):
        for k in range(NBUF):
            j = g * NBUF + k
            gather(j, k).wait()
            put(j, k).start()
            nj = j + NBUF - 1
            nbuf = (k + NBUF - 1) % NBUF

            @pl.when(nj < NCH)
            def _():
                @pl.when(j >= 1)
                def _():
                    put(j - 1, nbuf).wait()

                gather(nj, nbuf).start()

        return 0

    lax.fori_loop(0, NCH // NBUF, body, 0)
    put(NCH - 1, (NCH - 1) % NBUF).wait()


@functools.partial(
    pl.kernel,
    mesh=plsc.VectorSubcoreMesh(core_axis_name="c", subcore_axis_name="s"),
    out_type=jax.ShapeDtypeStruct((ROWS, C), jnp.float32),
    scratch_types=[
        pltpu.VMEM((XPW, N * B), jnp.int32),
        pltpu.VMEM((NBUF, CHUNK, C), jnp.float32),
        pltpu.VMEM_SHARED((TBL * TBL, C), jnp.float32),
        pltpu.SemaphoreType.DMA((NBUF,)),
        pltpu.SemaphoreType.DMA((NBUF,)),
    ],
)
def _sc_gather(table_hbm, idx_hbm, out_hbm, idx_v, rows_v, tbl_sh, gsem, psem):
    _gather_body(table_hbm, idx_hbm, out_hbm, idx_v, rows_v, tbl_sh, gsem, psem)


def kernel(i, p0, p1, center_offset):
    i = i.astype(jnp.int32)
    center_offset = center_offset.astype(jnp.int32)
    i0 = i[:, :, 0]
    i1 = i[:, :, 1]
    i0f = i0.reshape(1, N * B)
    i1f = i1.reshape(1, N * B)
    table, idx = _prep(i0, i1, i0f, i1f, p0, p1, center_offset)
    cm = _cm(i0, i1, i0f, i1f, center_offset)
    pe = _sc_gather(table, idx)
    pe = pe.reshape(N, N, B, C)
    cm = cm.reshape(N, N, B)
    return pe, cm
